# baseline (speedup 1.0000x reference)
import functools

import jax
import jax.numpy as jnp
from jax import lax
from jax.experimental import pallas as pl
from jax.experimental.pallas import tpu as pltpu
from jax.experimental.pallas import tpu_sc as plsc

N = 10000
F = 128
E = 320000
NLAYER = 3
BN_EPS = 1e-5

NS = 16
EPW = E // NS
EPWP = 20480
CK = 80
CH = EPWP // CK
IBC = 16
NB = CH // IBC
NGR = IBC // 4
NP = 10240
TRASH = 10100
RPS = NP // NS
ZR = 40


def _sc_agg_body(src_hbm, dst_hbm, h_hbm, out_hbm,
                 src_v, dst_v, r0, r1, r2, r3, agg_sh,
                 g0, g1, g2, g3, s0, s1, s2, s3):
    s = lax.axis_index("s")
    rows = (r0, r1, r2, r3)
    gsem = (g0, g1, g2, g3)
    ssem = (s0, s1, s2, s3)

    z16 = jnp.zeros((16,), jnp.float32)

    def _zrow(i, _):
        def _zcol(j, _):
            r0[i, pl.ds(j * 16, 16)] = z16
            return 0
        return lax.fori_loop(0, F // 16, _zcol, 0)

    lax.fori_loop(0, CK, _zrow, 0)
    for k in range(RPS // CK):
        pltpu.sync_copy(r0, agg_sh.at[pl.ds(s * RPS + k * CK, CK)])
    plsc.subcore_barrier()

    def _sidx(ch):
        return src_v.at[ch]

    def _gather(ch, k):
        pltpu.async_copy(h_hbm.at[_sidx(ch)], rows[k], gsem[k])

    def _gather_wait(ch, k):
        pltpu.make_async_copy(h_hbm.at[_sidx(ch)], rows[k], gsem[k]).wait()

    def _scatter(ch, k):
        pltpu.async_copy(rows[k], agg_sh.at[dst_v.at[ch]], ssem[k], add=True)

    def _scatter_wait(ch, k):
        pltpu.make_async_copy(rows[k], agg_sh.at[dst_v.at[ch]],
                              ssem[k]).wait()

    def _block(b, _):
        pltpu.sync_copy(src_hbm.at[s, b], src_v)
        pltpu.sync_copy(dst_hbm.at[s, b], dst_v)
        _gather(0, 0)
        _gather(1, 1)

        def _group(gr, _):
            for k in range(4):
                ch = 4 * gr + k
                _gather_wait(ch, k)
                _scatter(ch, k)
                kn = (k + 2) % 4
                if k < 2:
                    @pl.when(gr > 0)
                    def _():
                        _scatter_wait(ch - 2, kn)

                    _gather(ch + 2, kn)
                else:
                    @pl.when(gr < NGR - 1)
                    def _():
                        _scatter_wait(ch - 2, kn)
                        _gather(ch + 2, kn)
            return 0

        lax.fori_loop(0, NGR, _group, 0)
        for k in range(4):
            _scatter_wait(IBC - 4 + k, k)
        return 0

    lax.fori_loop(0, NB, _block, 0)
    plsc.subcore_barrier()

    pltpu.sync_copy(agg_sh.at[pl.ds(s * RPS, RPS)],
                    out_hbm.at[pl.ds(s * RPS, RPS)])


_sc_agg = functools.partial(
    pl.kernel,
    out_type=jax.ShapeDtypeStruct((NP, F), jnp.float32),
    mesh=plsc.VectorSubcoreMesh(core_axis_name="c", subcore_axis_name="s",
                                num_cores=1),
    scratch_types=(
        [pltpu.VMEM((IBC, CK), jnp.int32),
         pltpu.VMEM((IBC, CK), jnp.int32)]
        + [pltpu.VMEM((CK, F), jnp.float32)] * 4
        + [pltpu.VMEM_SHARED((NP, F), jnp.float32)]
        + [pltpu.SemaphoreType.DMA] * 8
    ),
)(_sc_agg_body)


BLK = 2000
G = N // BLK


def _stats_step(i, v, acc, st_ref):
    @pl.when(i == 0)
    def _():
        acc[...] = jnp.zeros_like(acc)

    acc[0:1, :] += jnp.sum(v, axis=0, keepdims=True)
    acc[1:2, :] += jnp.sum(v * v, axis=0, keepdims=True)

    @pl.when(i == G - 1)
    def _():
        st_ref[...] = acc[...]


def _transform_body(x_ref, w_ref, b_ref, y_ref, st_ref, acc):
    y = jnp.dot(x_ref[...], w_ref[...], preferred_element_type=jnp.float32)
    y = y + b_ref[...]
    y_ref[...] = y
    _stats_step(pl.program_id(0), y, acc, st_ref)


def _mlp_body(h_ref, p_ref, w1_ref, w2_ref, u_ref, st_ref, acc):
    m = h_ref[...] + p_ref[...]
    t = jnp.maximum(jnp.dot(m, w1_ref[...], preferred_element_type=jnp.float32), 0.0)
    u = jnp.maximum(jnp.dot(t, w2_ref[...], preferred_element_type=jnp.float32), 0.0)
    u_ref[...] = u
    _stats_step(pl.program_id(0), u, acc, st_ref)


def _bn_body(y_ref, st_ref, g_ref, b_ref, o_ref):
    mu = st_ref[0:1, :] * (1.0 / N)
    var = st_ref[1:2, :] * (1.0 / N) - mu * mu
    inv = lax.rsqrt(var + BN_EPS) * g_ref[...]
    o_ref[...] = (y_ref[...] - mu) * inv + b_ref[...]


_row_spec = pl.BlockSpec((BLK, F), lambda i: (i, 0))
_full_spec = pl.BlockSpec((F, F), lambda i: (0, 0))
_st_spec = pl.BlockSpec((2, F), lambda i: (0, 0))
_vec_spec = pl.BlockSpec((1, F), lambda i: (0, 0))

_nf_shape = jax.ShapeDtypeStruct((N, F), jnp.float32)
_st_shape = jax.ShapeDtypeStruct((2, F), jnp.float32)

_transform = pl.pallas_call(
    _transform_body,
    grid=(G,),
    in_specs=[_row_spec, _full_spec, _vec_spec],
    out_specs=[_row_spec, _st_spec],
    out_shape=[_nf_shape, _st_shape],
    scratch_shapes=[pltpu.VMEM((2, F), jnp.float32)],
)

_mlp = pl.pallas_call(
    _mlp_body,
    grid=(G,),
    in_specs=[_row_spec, _row_spec, _full_spec, _full_spec],
    out_specs=[_row_spec, _st_spec],
    out_shape=[_nf_shape, _st_shape],
    scratch_shapes=[pltpu.VMEM((2, F), jnp.float32)],
)

_bn = pl.pallas_call(
    _bn_body,
    grid=(G,),
    in_specs=[_row_spec, _st_spec, _vec_spec, _vec_spec],
    out_specs=_row_spec,
    out_shape=_nf_shape,
)


def kernel(x, edge_index, batch, Wt, bt, bn0_g, bn0_b, W1, W2, bng, bnb):
    pad_dst = N + (jnp.arange(EPWP - EPW, dtype=jnp.int32) % (NP - N))
    pad_src = jnp.arange(EPW, EPWP, dtype=jnp.int32) % N
    src = jnp.concatenate(
        [edge_index[0].reshape(NS, EPW),
         jnp.broadcast_to(pad_src, (NS, EPWP - EPW))],
        axis=1).reshape(NS, NB, IBC, CK)
    dst = jnp.concatenate(
        [edge_index[1].reshape(NS, EPW),
         jnp.broadcast_to(pad_dst, (NS, EPWP - EPW))],
        axis=1).reshape(NS, NB, IBC, CK)

    y, st = _transform(x, Wt, bt.reshape(1, F))
    h = _bn(y, st, bn0_g.reshape(1, F), bn0_b.reshape(1, F))
    for i in range(NLAYER):
        parts = _sc_agg(src, dst, h)
        u, st = _mlp(h, parts, W1[i], W2[i])
        h = _bn(u, st, bng[i].reshape(1, F), bnb[i].reshape(1, F))
    return h


# speedup vs baseline: 1.0039x; 1.0039x over previous
import functools

import jax
import jax.numpy as jnp
from jax import lax
from jax.experimental import pallas as pl
from jax.experimental.pallas import tpu as pltpu
from jax.experimental.pallas import tpu_sc as plsc

N = 10000
F = 128
E = 320000
NLAYER = 3
BN_EPS = 1e-5

NS = 16
EPW = E // NS
EPWP = 20480
CK = 64
CH = EPWP // CK
IBC = 32
NB = CH // IBC
NGR = IBC // 4
NP = 10240
TRASH = 10100
RPS = NP // NS
ZR = 40


def _sc_agg_body(src_hbm, dst_hbm, h_hbm, out_hbm,
                 src_v, dst_v, r0, r1, r2, r3, agg_sh,
                 g0, g1, g2, g3, s0, s1, s2, s3):
    s = lax.axis_index("s")
    rows = (r0, r1, r2, r3)
    gsem = (g0, g1, g2, g3)
    ssem = (s0, s1, s2, s3)

    z16 = jnp.zeros((16,), jnp.float32)

    def _zrow(i, _):
        def _zcol(j, _):
            r0[i, pl.ds(j * 16, 16)] = z16
            return 0
        return lax.fori_loop(0, F // 16, _zcol, 0)

    lax.fori_loop(0, CK, _zrow, 0)
    for k in range(RPS // CK):
        pltpu.sync_copy(r0, agg_sh.at[pl.ds(s * RPS + k * CK, CK)])
    plsc.subcore_barrier()

    def _sidx(ch):
        return src_v.at[ch]

    def _gather(ch, k):
        pltpu.async_copy(h_hbm.at[_sidx(ch)], rows[k], gsem[k])

    def _gather_wait(ch, k):
        pltpu.make_async_copy(h_hbm.at[_sidx(ch)], rows[k], gsem[k]).wait()

    def _scatter(ch, k):
        pltpu.async_copy(rows[k], agg_sh.at[dst_v.at[ch]], ssem[k], add=True)

    def _scatter_wait(ch, k):
        pltpu.make_async_copy(rows[k], agg_sh.at[dst_v.at[ch]],
                              ssem[k]).wait()

    def _block(b, _):
        pltpu.sync_copy(src_hbm.at[s, b], src_v)
        pltpu.sync_copy(dst_hbm.at[s, b], dst_v)
        _gather(0, 0)
        _gather(1, 1)

        def _group(gr, _):
            for k in range(4):
                ch = 4 * gr + k
                _gather_wait(ch, k)
                _scatter(ch, k)
                kn = (k + 2) % 4
                if k < 2:
                    @pl.when(gr > 0)
                    def _():
                        _scatter_wait(ch - 2, kn)

                    _gather(ch + 2, kn)
                else:
                    @pl.when(gr < NGR - 1)
                    def _():
                        _scatter_wait(ch - 2, kn)
                        _gather(ch + 2, kn)
            return 0

        lax.fori_loop(0, NGR, _group, 0)
        for k in range(4):
            _scatter_wait(IBC - 4 + k, k)
        return 0

    lax.fori_loop(0, NB, _block, 0)
    plsc.subcore_barrier()

    pltpu.sync_copy(agg_sh.at[pl.ds(s * RPS, RPS)],
                    out_hbm.at[pl.ds(s * RPS, RPS)])


_sc_agg = functools.partial(
    pl.kernel,
    out_type=jax.ShapeDtypeStruct((NP, F), jnp.float32),
    mesh=plsc.VectorSubcoreMesh(core_axis_name="c", subcore_axis_name="s",
                                num_cores=1),
    scratch_types=(
        [pltpu.VMEM((IBC, CK), jnp.int32),
         pltpu.VMEM((IBC, CK), jnp.int32)]
        + [pltpu.VMEM((CK, F), jnp.float32)] * 4
        + [pltpu.VMEM_SHARED((NP, F), jnp.float32)]
        + [pltpu.SemaphoreType.DMA] * 8
    ),
)(_sc_agg_body)


BLK = 2000
G = N // BLK


def _stats_step(i, v, acc, st_ref):
    @pl.when(i == 0)
    def _():
        acc[...] = jnp.zeros_like(acc)

    acc[0:1, :] += jnp.sum(v, axis=0, keepdims=True)
    acc[1:2, :] += jnp.sum(v * v, axis=0, keepdims=True)

    @pl.when(i == G - 1)
    def _():
        st_ref[...] = acc[...]


def _transform_body(x_ref, w_ref, b_ref, y_ref, st_ref, acc):
    y = jnp.dot(x_ref[...], w_ref[...], preferred_element_type=jnp.float32)
    y = y + b_ref[...]
    y_ref[...] = y
    _stats_step(pl.program_id(0), y, acc, st_ref)


def _mlp_body(h_ref, p_ref, w1_ref, w2_ref, u_ref, st_ref, acc):
    m = h_ref[...] + p_ref[...]
    t = jnp.maximum(jnp.dot(m, w1_ref[...], preferred_element_type=jnp.float32), 0.0)
    u = jnp.maximum(jnp.dot(t, w2_ref[...], preferred_element_type=jnp.float32), 0.0)
    u_ref[...] = u
    _stats_step(pl.program_id(0), u, acc, st_ref)


def _bn_body(y_ref, st_ref, g_ref, b_ref, o_ref):
    mu = st_ref[0:1, :] * (1.0 / N)
    var = st_ref[1:2, :] * (1.0 / N) - mu * mu
    inv = lax.rsqrt(var + BN_EPS) * g_ref[...]
    o_ref[...] = (y_ref[...] - mu) * inv + b_ref[...]


_row_spec = pl.BlockSpec((BLK, F), lambda i: (i, 0))
_full_spec = pl.BlockSpec((F, F), lambda i: (0, 0))
_st_spec = pl.BlockSpec((2, F), lambda i: (0, 0))
_vec_spec = pl.BlockSpec((1, F), lambda i: (0, 0))

_nf_shape = jax.ShapeDtypeStruct((N, F), jnp.float32)
_st_shape = jax.ShapeDtypeStruct((2, F), jnp.float32)

_transform = pl.pallas_call(
    _transform_body,
    grid=(G,),
    in_specs=[_row_spec, _full_spec, _vec_spec],
    out_specs=[_row_spec, _st_spec],
    out_shape=[_nf_shape, _st_shape],
    scratch_shapes=[pltpu.VMEM((2, F), jnp.float32)],
)

_mlp = pl.pallas_call(
    _mlp_body,
    grid=(G,),
    in_specs=[_row_spec, _row_spec, _full_spec, _full_spec],
    out_specs=[_row_spec, _st_spec],
    out_shape=[_nf_shape, _st_shape],
    scratch_shapes=[pltpu.VMEM((2, F), jnp.float32)],
)

_bn = pl.pallas_call(
    _bn_body,
    grid=(G,),
    in_specs=[_row_spec, _st_spec, _vec_spec, _vec_spec],
    out_specs=_row_spec,
    out_shape=_nf_shape,
)


def kernel(x, edge_index, batch, Wt, bt, bn0_g, bn0_b, W1, W2, bng, bnb):
    pad_dst = N + (jnp.arange(EPWP - EPW, dtype=jnp.int32) % (NP - N))
    pad_src = jnp.arange(EPW, EPWP, dtype=jnp.int32) % N
    src = jnp.concatenate(
        [edge_index[0].reshape(NS, EPW),
         jnp.broadcast_to(pad_src, (NS, EPWP - EPW))],
        axis=1).reshape(NS, NB, IBC, CK)
    dst = jnp.concatenate(
        [edge_index[1].reshape(NS, EPW),
         jnp.broadcast_to(pad_dst, (NS, EPWP - EPW))],
        axis=1).reshape(NS, NB, IBC, CK)

    y, st = _transform(x, Wt, bt.reshape(1, F))
    h = _bn(y, st, bn0_g.reshape(1, F), bn0_b.reshape(1, F))
    for i in range(NLAYER):
        parts = _sc_agg(src, dst, h)
        u, st = _mlp(h, parts, W1[i], W2[i])
        h = _bn(u, st, bng[i].reshape(1, F), bnb[i].reshape(1, F))
    return h


# speedup vs baseline: 1.0214x; 1.0175x over previous
import functools

import jax
import jax.numpy as jnp
from jax import lax
from jax.experimental import pallas as pl
from jax.experimental.pallas import tpu as pltpu
from jax.experimental.pallas import tpu_sc as plsc

N = 10000
F = 128
E = 320000
NLAYER = 3
BN_EPS = 1e-5

NS = 16
EPW = E // NS
EPWP = 20480
CK = 64
CH = EPWP // CK
IBC = 40
NB = CH // IBC
NGR = IBC // 4
NP = 10240
TRASH = 10100
RPS = NP // NS
ZR = 40


def _sc_agg_body(src_hbm, dst_hbm, h_hbm, out_hbm,
                 src_v, dst_v, r0, r1, r2, r3, agg_sh,
                 g0, g1, g2, g3, s0, s1, s2, s3):
    s = lax.axis_index("s")
    rows = (r0, r1, r2, r3)
    gsem = (g0, g1, g2, g3)
    ssem = (s0, s1, s2, s3)

    z16 = jnp.zeros((16,), jnp.float32)

    def _zrow(i, _):
        def _zcol(j, _):
            r0[i, pl.ds(j * 16, 16)] = z16
            return 0
        return lax.fori_loop(0, F // 16, _zcol, 0)

    lax.fori_loop(0, CK, _zrow, 0)
    for k in range(RPS // CK):
        pltpu.sync_copy(r0, agg_sh.at[pl.ds(s * RPS + k * CK, CK)])
    plsc.subcore_barrier()

    def _sidx(ch):
        return src_v.at[ch]

    def _gather(ch, k):
        pltpu.async_copy(h_hbm.at[_sidx(ch)], rows[k], gsem[k])

    def _gather_wait(ch, k):
        pltpu.make_async_copy(h_hbm.at[_sidx(ch)], rows[k], gsem[k]).wait()

    def _scatter(ch, k):
        pltpu.async_copy(rows[k], agg_sh.at[dst_v.at[ch]], ssem[k], add=True)

    def _scatter_wait(ch, k):
        pltpu.make_async_copy(rows[k], agg_sh.at[dst_v.at[ch]],
                              ssem[k]).wait()

    def _block(b, _):
        pltpu.sync_copy(src_hbm.at[s, b], src_v)
        pltpu.sync_copy(dst_hbm.at[s, b], dst_v)
        _gather(0, 0)
        _gather(1, 1)

        def _group(gr, _):
            for k in range(4):
                ch = 4 * gr + k
                _gather_wait(ch, k)
                _scatter(ch, k)
                kn = (k + 2) % 4
                if k < 2:
                    @pl.when(gr > 0)
                    def _():
                        _scatter_wait(ch - 2, kn)

                    _gather(ch + 2, kn)
                else:
                    @pl.when(gr < NGR - 1)
                    def _():
                        _scatter_wait(ch - 2, kn)
                        _gather(ch + 2, kn)
            return 0

        lax.fori_loop(0, NGR, _group, 0)
        for k in range(4):
            _scatter_wait(IBC - 4 + k, k)
        return 0

    lax.fori_loop(0, NB, _block, 0)
    plsc.subcore_barrier()

    pltpu.sync_copy(agg_sh.at[pl.ds(s * RPS, RPS)],
                    out_hbm.at[pl.ds(s * RPS, RPS)])


_sc_agg = functools.partial(
    pl.kernel,
    out_type=jax.ShapeDtypeStruct((NP, F), jnp.float32),
    mesh=plsc.VectorSubcoreMesh(core_axis_name="c", subcore_axis_name="s",
                                num_cores=1),
    scratch_types=(
        [pltpu.VMEM((IBC, CK), jnp.int32),
         pltpu.VMEM((IBC, CK), jnp.int32)]
        + [pltpu.VMEM((CK, F), jnp.float32)] * 4
        + [pltpu.VMEM_SHARED((NP, F), jnp.float32)]
        + [pltpu.SemaphoreType.DMA] * 8
    ),
)(_sc_agg_body)


BLK = 2000
G = N // BLK


def _stats_step(i, v, acc, st_ref):
    @pl.when(i == 0)
    def _():
        acc[...] = jnp.zeros_like(acc)

    acc[0:1, :] += jnp.sum(v, axis=0, keepdims=True)
    acc[1:2, :] += jnp.sum(v * v, axis=0, keepdims=True)

    @pl.when(i == G - 1)
    def _():
        st_ref[...] = acc[...]


def _transform_body(x_ref, w_ref, b_ref, y_ref, st_ref, acc):
    y = jnp.dot(x_ref[...], w_ref[...], preferred_element_type=jnp.float32)
    y = y + b_ref[...]
    y_ref[...] = y
    _stats_step(pl.program_id(0), y, acc, st_ref)


def _mlp_body(h_ref, p_ref, w1_ref, w2_ref, u_ref, st_ref, acc):
    m = h_ref[...] + p_ref[...]
    t = jnp.maximum(jnp.dot(m, w1_ref[...], preferred_element_type=jnp.float32), 0.0)
    u = jnp.maximum(jnp.dot(t, w2_ref[...], preferred_element_type=jnp.float32), 0.0)
    u_ref[...] = u
    _stats_step(pl.program_id(0), u, acc, st_ref)


def _bn_body(y_ref, st_ref, g_ref, b_ref, o_ref):
    mu = st_ref[0:1, :] * (1.0 / N)
    var = st_ref[1:2, :] * (1.0 / N) - mu * mu
    inv = lax.rsqrt(var + BN_EPS) * g_ref[...]
    o_ref[...] = (y_ref[...] - mu) * inv + b_ref[...]


_row_spec = pl.BlockSpec((BLK, F), lambda i: (i, 0))
_full_spec = pl.BlockSpec((F, F), lambda i: (0, 0))
_st_spec = pl.BlockSpec((2, F), lambda i: (0, 0))
_vec_spec = pl.BlockSpec((1, F), lambda i: (0, 0))

_nf_shape = jax.ShapeDtypeStruct((N, F), jnp.float32)
_st_shape = jax.ShapeDtypeStruct((2, F), jnp.float32)

_transform = pl.pallas_call(
    _transform_body,
    grid=(G,),
    in_specs=[_row_spec, _full_spec, _vec_spec],
    out_specs=[_row_spec, _st_spec],
    out_shape=[_nf_shape, _st_shape],
    scratch_shapes=[pltpu.VMEM((2, F), jnp.float32)],
)

_mlp = pl.pallas_call(
    _mlp_body,
    grid=(G,),
    in_specs=[_row_spec, _row_spec, _full_spec, _full_spec],
    out_specs=[_row_spec, _st_spec],
    out_shape=[_nf_shape, _st_shape],
    scratch_shapes=[pltpu.VMEM((2, F), jnp.float32)],
)

_bn = pl.pallas_call(
    _bn_body,
    grid=(G,),
    in_specs=[_row_spec, _st_spec, _vec_spec, _vec_spec],
    out_specs=_row_spec,
    out_shape=_nf_shape,
)


def kernel(x, edge_index, batch, Wt, bt, bn0_g, bn0_b, W1, W2, bng, bnb):
    pad_dst = N + (jnp.arange(EPWP - EPW, dtype=jnp.int32) % (NP - N))
    pad_src = jnp.arange(EPW, EPWP, dtype=jnp.int32) % N
    src = jnp.concatenate(
        [edge_index[0].reshape(NS, EPW),
         jnp.broadcast_to(pad_src, (NS, EPWP - EPW))],
        axis=1).reshape(NS, NB, IBC, CK)
    dst = jnp.concatenate(
        [edge_index[1].reshape(NS, EPW),
         jnp.broadcast_to(pad_dst, (NS, EPWP - EPW))],
        axis=1).reshape(NS, NB, IBC, CK)

    y, st = _transform(x, Wt, bt.reshape(1, F))
    h = _bn(y, st, bn0_g.reshape(1, F), bn0_b.reshape(1, F))
    for i in range(NLAYER):
        parts = _sc_agg(src, dst, h)
        u, st = _mlp(h, parts, W1[i], W2[i])
        h = _bn(u, st, bng[i].reshape(1, F), bnb[i].reshape(1, F))
    return h
